# baseline (speedup 1.0000x reference)
import functools

import jax
import jax.numpy as jnp
from jax import lax
from jax.experimental import pallas as pl
from jax.experimental.pallas import tpu as pltpu
from jax.experimental.pallas import tpu_sc as plsc

D = 64
B = 4096
L = 200
NW = 32
LANE = 128
DG = D // 8
NBUF = 4
OB = 2

_mesh = plsc.VectorSubcoreMesh(core_axis_name="c", subcore_axis_name="s")


@functools.partial(
    pl.kernel,
    mesh=_mesh,
    out_type=(
        jax.ShapeDtypeStruct((L, DG, NW, 8, LANE), jnp.float32),
        jax.ShapeDtypeStruct((L, DG, NW, 8, LANE), jnp.float32),
    ),
    scratch_types=[
        pltpu.VMEM((L, LANE), jnp.int32),
        pltpu.VMEM((NBUF, LANE, D), jnp.float32),
        pltpu.VMEM((OB, DG, 8, LANE), jnp.float32),
    ]
    + [pltpu.SemaphoreType.DMA] * (NBUF + OB),
    compiler_params=pltpu.CompilerParams(
        use_tc_tiling_on_sc=False, needs_layout_passes=False
    ),
)
def _embed2(n2v, t2v, seq_i, ts_i, out_x, out_t, idx_v, awide, bpack, *sems):
    wid = lax.axis_index("s") * 2 + lax.axis_index("c")
    gs, os_ = sems[:NBUF], sems[NBUF:]
    iota = lax.iota(jnp.int32, 16)
    rowidx = [iota + bb * 16 for bb in range(LANE // 16)]
    for table, idx_hbm, out_hbm in ((n2v, seq_i, out_x), (t2v, ts_i, out_t)):
        pltpu.sync_copy(idx_hbm.at[wid], idx_v)
        for b in range(NBUF - 1):
            pltpu.async_copy(table.at[idx_v.at[b]], awide.at[b], gs[b])

        def body(g, _, table=table, out_hbm=out_hbm):
            for b in range(NBUF):
                c = g * NBUF + b
                o = b % OB
                pltpu.make_async_copy(
                    table.at[idx_v.at[c]], awide.at[b], gs[b]
                ).wait()
                def recycle(c=c, o=o, out_hbm=out_hbm):
                    for dg in range(DG):
                        pltpu.make_async_copy(
                            bpack.at[o].at[dg],
                            out_hbm.at[c - OB].at[dg].at[wid],
                            os_[o],
                        ).wait()
                if b < OB:
                    pl.when(g > 0)(recycle)
                else:
                    recycle()
                def trow(d, col, b=b, o=o):
                    dg = d // 8
                    dsub = d - dg * 8
                    for bb in range(LANE // 16):
                        v = plsc.load_gather(awide.at[b], [rowidx[bb], col])
                        bpack[o, dg, dsub, pl.ds(bb * 16, 16)] = v
                    return col + 1
                lax.fori_loop(0, D, trow, iota * 0)
                nb = (b + NBUF - 1) % NBUF
                nxt = jnp.minimum(c + NBUF - 1, L - 1)
                pltpu.async_copy(table.at[idx_v.at[nxt]], awide.at[nb], gs[nb])
                for dg in range(DG):
                    pltpu.async_copy(
                        bpack.at[o].at[dg],
                        out_hbm.at[c].at[dg].at[wid],
                        os_[o],
                    )
            return ()

        lax.fori_loop(0, L // NBUF, body, ())
        for b in range(NBUF - 1):
            pltpu.make_async_copy(
                table.at[idx_v.at[L - 1]], awide.at[b], gs[b]
            ).wait()
        for c in range(L - OB, L):
            for dg in range(DG):
                pltpu.make_async_copy(
                    bpack.at[c % OB].at[dg],
                    out_hbm.at[c].at[dg].at[wid],
                    os_[c % OB],
                ).wait()


def kernel(seq, ts, node2vec, time2vec):
    seq_r = seq.astype(jnp.int32).reshape(NW, LANE, L).transpose(0, 2, 1)
    ts_r = ts.astype(jnp.int32).reshape(NW, LANE, L).transpose(0, 2, 1)
    x5, t5 = _embed2(node2vec, time2vec, seq_r, ts_r)
    x = x5.transpose(2, 4, 0, 1, 3).reshape(B, L, D)
    t = t5.transpose(2, 4, 0, 1, 3).reshape(B, L, D)
    return x, t


# speedup vs baseline: 2.8146x; 2.8146x over previous
import functools

import jax
import jax.numpy as jnp
from jax import lax
from jax.experimental import pallas as pl
from jax.experimental.pallas import tpu as pltpu
from jax.experimental.pallas import tpu_sc as plsc

D = 64
B = 4096
L = 200
NW = 32
LANE = 128
DG = D // 8
NBUF = 4
OB = 2
PITCH = 129

_mesh = plsc.VectorSubcoreMesh(core_axis_name="c", subcore_axis_name="s")


@functools.partial(
    pl.kernel,
    mesh=_mesh,
    out_type=(
        jax.ShapeDtypeStruct((L, DG, NW, 8, LANE), jnp.float32),
        jax.ShapeDtypeStruct((L, DG, NW, 8, LANE), jnp.float32),
    ),
    scratch_types=[
        pltpu.VMEM((L, LANE), jnp.int32),
        pltpu.VMEM((NBUF, LANE, D), jnp.float32),
        pltpu.VMEM((OB, DG, 8, PITCH), jnp.float32),
    ]
    + [pltpu.SemaphoreType.DMA] * (NBUF + OB),
    compiler_params=pltpu.CompilerParams(
        use_tc_tiling_on_sc=False, needs_layout_passes=False
    ),
)
def _embed2(n2v, t2v, seq_i, ts_i, out_x, out_t, idx_v, awide, bpack, *sems):
    wid = lax.axis_index("s") * 2 + lax.axis_index("c")
    gs, os_ = sems[:NBUF], sems[NBUF:]
    iota = lax.iota(jnp.int32, 16)
    dgidx = [(iota + j * 16) >> 3 for j in range(D // 16)]
    dsubidx = [(iota + j * 16) & 7 for j in range(D // 16)]
    for table, idx_hbm, out_hbm in ((n2v, seq_i, out_x), (t2v, ts_i, out_t)):
        pltpu.sync_copy(idx_hbm.at[wid], idx_v)
        for b in range(NBUF - 1):
            pltpu.async_copy(table.at[idx_v.at[b]], awide.at[b], gs[b])

        def body(g, _, table=table, out_hbm=out_hbm):
            for b in range(NBUF):
                c = g * NBUF + b
                o = b % OB
                pltpu.make_async_copy(
                    table.at[idx_v.at[c]], awide.at[b], gs[b]
                ).wait()
                def recycle(c=c, o=o, out_hbm=out_hbm):
                    for dg in range(DG):
                        pltpu.make_async_copy(
                            bpack.at[o].at[dg].at[:, pl.ds(0, LANE)],
                            out_hbm.at[c - OB].at[dg].at[wid],
                            os_[o],
                        ).wait()
                if b < OB:
                    pl.when(g > 0)(recycle)
                else:
                    recycle()
                def trow(r, rv, b=b, o=o):
                    for j in range(D // 16):
                        v = awide[b, r, pl.ds(j * 16, 16)]
                        plsc.store_scatter(
                            bpack.at[o], [dgidx[j], dsubidx[j], rv], v
                        )
                    return rv + 1
                lax.fori_loop(0, LANE, trow, iota * 0)
                nb = (b + NBUF - 1) % NBUF
                nxt = jnp.minimum(c + NBUF - 1, L - 1)
                pltpu.async_copy(table.at[idx_v.at[nxt]], awide.at[nb], gs[nb])
                for dg in range(DG):
                    pltpu.async_copy(
                        bpack.at[o].at[dg].at[:, pl.ds(0, LANE)],
                        out_hbm.at[c].at[dg].at[wid],
                        os_[o],
                    )
            return ()

        lax.fori_loop(0, L // NBUF, body, ())
        for b in range(NBUF - 1):
            pltpu.make_async_copy(
                table.at[idx_v.at[L - 1]], awide.at[b], gs[b]
            ).wait()
        for c in range(L - OB, L):
            for dg in range(DG):
                pltpu.make_async_copy(
                    bpack.at[c % OB].at[dg].at[:, pl.ds(0, LANE)],
                    out_hbm.at[c].at[dg].at[wid],
                    os_[c % OB],
                ).wait()


def kernel(seq, ts, node2vec, time2vec):
    seq_r = seq.astype(jnp.int32).reshape(NW, LANE, L).transpose(0, 2, 1)
    ts_r = ts.astype(jnp.int32).reshape(NW, LANE, L).transpose(0, 2, 1)
    x5, t5 = _embed2(node2vec, time2vec, seq_r, ts_r)
    x = x5.transpose(2, 4, 0, 1, 3).reshape(B, L, D)
    t = t5.transpose(2, 4, 0, 1, 3).reshape(B, L, D)
    return x, t
